# baseline (speedup 1.0000x reference)
import jax
import jax.numpy as jnp
from jax.experimental import pallas as pl

_EPS = 1e-08
_BLOCK = 512


def _router_kernel(h_ref, a_ref, out_ref):
    a = a_ref[...]
    a_n = a / jnp.maximum(
        jnp.sqrt(jnp.sum(a * a, axis=1, keepdims=True)), _EPS)
    hb = h_ref[...]
    h_n = hb / jnp.maximum(
        jnp.sqrt(jnp.sum(hb * hb, axis=1, keepdims=True)), _EPS)
    scores = jax.lax.dot_general(
        h_n, a_n, (((1,), (1,)), ((), ())),
        preferred_element_type=jnp.float32)
    idx = jnp.argmax(scores, axis=1).astype(jnp.int32)
    out_ref[...] = idx[:, None]


def kernel(h, A):
    B, D = h.shape
    K = A.shape[0]
    return pl.pallas_call(
        _router_kernel,
        grid=(B // _BLOCK,),
        in_specs=[
            pl.BlockSpec((_BLOCK, D), lambda i: (i, 0)),
            pl.BlockSpec((K, D), lambda i: (0, 0)),
        ],
        out_specs=pl.BlockSpec((_BLOCK, 1), lambda i: (i, 0)),
        out_shape=jax.ShapeDtypeStruct((B, 1), jnp.int32),
    )(h, A)


# speedup vs baseline: 1.2229x; 1.2229x over previous
import jax
import jax.numpy as jnp
from jax.experimental import pallas as pl

_EPS = 1e-08
_BLOCK = 2048


def _router_kernel(h_ref, a_ref, out_ref):
    a = a_ref[...]
    a_n = a / jnp.maximum(
        jnp.sqrt(jnp.sum(a * a, axis=1, keepdims=True)), _EPS)
    hb = h_ref[...]
    h_n = hb / jnp.maximum(
        jnp.sqrt(jnp.sum(hb * hb, axis=1, keepdims=True)), _EPS)
    scores = jax.lax.dot_general(
        h_n, a_n, (((1,), (1,)), ((), ())),
        preferred_element_type=jnp.float32)
    idx = jnp.argmax(scores, axis=1).astype(jnp.int32)
    out_ref[...] = idx[:, None]


def kernel(h, A):
    B, D = h.shape
    K = A.shape[0]
    return pl.pallas_call(
        _router_kernel,
        grid=(B // _BLOCK,),
        in_specs=[
            pl.BlockSpec((_BLOCK, D), lambda i: (i, 0)),
            pl.BlockSpec((K, D), lambda i: (0, 0)),
        ],
        out_specs=pl.BlockSpec((_BLOCK, 1), lambda i: (i, 0)),
        out_shape=jax.ShapeDtypeStruct((B, 1), jnp.int32),
    )(h, A)
